# baseline (speedup 1.0000x reference)
import jax
import jax.numpy as jnp
from jax import lax
from jax.experimental import pallas as pl
from jax.experimental.pallas import tpu as pltpu
from jax.experimental.pallas import tpu_sc as plsc

_K = 2048
_N = 32768
_B = 128
_L = 16
_NV = _N // _L
_NC = 2
_NS = 16
_NW = _NC * _NS
_NSTRIPE = 8
_HSTRIDE = 2048
_TC_ROWS = 64
_SC_ROWS = _B - _TC_ROWS
_RPW = _SC_ROWS // _NW


def _sc_body(x_hbm, out_hbm, xa_v, xb_v, y_v, hist_v, s_v, sem_in, sem_out):
    cid = lax.axis_index("c")
    sid = lax.axis_index("s")
    wid = sid * _NC + cid
    lanes = lax.broadcasted_iota(jnp.int32, (_L,), 0)
    stripe_base = (lanes & 7) * _HSTRIDE
    addend = jnp.where(lanes < 8, jnp.int32(1), jnp.int32(1 << 16))
    zero16 = jnp.zeros((_L,), jnp.int32)
    absmask = jnp.int32(0x7FFFFFFF)
    bufs = (xa_v, xb_v)

    @plsc.parallel_loop(0, _NSTRIPE * (_HSTRIDE // _L), unroll=8)
    def _(t):
        hist_v[pl.ds(t * _L, _L)] = zero16

    in_cp = pltpu.async_copy(x_hbm.at[wid * _RPW], bufs[0], sem_in)
    out_cp = None

    for r in range(_RPW):
        row = wid * _RPW + r
        x_v = bufs[r % 2]
        in_cp.wait()
        if r + 1 < _RPW:
            in_cp = pltpu.async_copy(
                x_hbm.at[row + 1], bufs[(r + 1) % 2], sem_in)

        def u_of(j, x_v=x_v):
            x = x_v[pl.ds(j * _L, _L)]
            return x, lax.bitcast_convert_type(x, jnp.int32) & absmask

        def suffix_pick(nbuckets, rk):
            nch = nbuckets // _L

            @plsc.parallel_loop(
                0, nch, unroll=4, carry=(jnp.int32(0), zero16))
            def carry_out(jj, carry, nch=nch):
                c = nch - 1 - jj
                csum, cnt = carry
                acc = hist_v[pl.ds(0 * _HSTRIDE + c * _L, _L)]
                hist_v[pl.ds(0 * _HSTRIDE + c * _L, _L)] = zero16
                for l in range(1, _NSTRIPE):
                    acc = acc + hist_v[pl.ds(l * _HSTRIDE + c * _L, _L)]
                    hist_v[pl.ds(l * _HSTRIDE + c * _L, _L)] = zero16
                tot = (acc & jnp.int32(0xFFFF)) + (acc >> 16)
                s = plsc.cumsum(lax.rev(tot, dimensions=(0,)))
                s_desc = s + csum
                cnt = cnt + plsc.all_reduce_population_count(s_desc >= rk)
                s_v[pl.ds(c * _L, _L)] = lax.rev(s_desc, dimensions=(0,))
                return csum + jnp.sum(tot), cnt

            _, cntv = carry_out
            b = jnp.max(cntv) - 1
            s_v[pl.ds(nbuckets, _L)] = zero16
            above = jnp.max(plsc.load_gather(s_v, [zero16 + (b + 1)]))
            return b, above

        @plsc.parallel_loop(0, _NV, unroll=8)
        def _(j):
            _, u = u_of(j)
            plsc.addupdate_scatter(
                hist_v, [stripe_base + (u >> 20)], addend)

        b1, above1 = suffix_pick(2048, jnp.int32(_K))
        rk = jnp.int32(_K) - above1
        pref = b1 << 20

        @plsc.parallel_loop(0, _NV, unroll=8)
        def _(j, b1=b1):
            _, u = u_of(j)
            m = (u >> 20) == b1
            plsc.addupdate_scatter(
                hist_v, [stripe_base + ((u >> 10) & jnp.int32(1023))],
                addend, mask=m)

        b2, above2 = suffix_pick(1024, rk)
        rk = rk - above2
        pref = pref | (b2 << 10)

        @plsc.parallel_loop(0, _NV, unroll=8)
        def _(j, pref=pref):
            _, u = u_of(j)
            m = (u >> 10) == (pref >> 10)
            plsc.addupdate_scatter(
                hist_v, [stripe_base + (u & jnp.int32(1023))],
                addend, mask=m)

        b3, _ = suffix_pick(1024, rk)
        thr = pref | b3
        if out_cp is not None:
            out_cp.wait()

        @plsc.parallel_loop(0, _NV, unroll=8)
        def _(j, thr=thr):
            x, u = u_of(j)
            y_v[pl.ds(j * _L, _L)] = jnp.where(u >= thr, x, jnp.float32(0.0))

        out_cp = pltpu.async_copy(y_v, out_hbm.at[row], sem_out)

    out_cp.wait()


def _sc_part(x):
    mesh = plsc.VectorSubcoreMesh(
        core_axis_name="c", subcore_axis_name="s",
        num_cores=_NC, num_subcores=_NS)
    return pl.kernel(
        _sc_body,
        out_type=jax.ShapeDtypeStruct((_SC_ROWS, _N), jnp.float32),
        mesh=mesh,
        scratch_types=[
            pltpu.VMEM((_N,), jnp.float32),
            pltpu.VMEM((_N,), jnp.float32),
            pltpu.VMEM((_N,), jnp.float32),
            pltpu.VMEM((_NSTRIPE * _HSTRIDE,), jnp.int32),
            pltpu.VMEM((2048 + _L,), jnp.int32),
            pltpu.SemaphoreType.DMA,
            pltpu.SemaphoreType.DMA,
        ],
        compiler_params=pltpu.CompilerParams(needs_layout_passes=False),
    )(x)


def _tc_block(x_ref, o_ref):
    x = x_ref[...]
    u = lax.bitcast_convert_type(x, jnp.int32) & jnp.int32(0x7FFFFFFF)

    def step(_, carry):
        lo, hi = carry
        mid = lo + ((hi - lo) >> 1)
        cnt = jnp.sum((u >= mid).astype(jnp.int32), axis=1, keepdims=True)
        ge = cnt >= _K
        return jnp.where(ge, mid, lo), jnp.where(ge, hi, mid)

    rows = x.shape[0]
    lo0 = jnp.zeros((rows, 1), jnp.int32)
    hi0 = jnp.full((rows, 1), 0x7F800000, jnp.int32)
    lo, _ = lax.fori_loop(0, 31, step, (lo0, hi0))
    o_ref[...] = jnp.where(u >= lo, x, jnp.float32(0.0))


def _tc_part(x):
    rows_per_block = 16
    grid = _TC_ROWS // rows_per_block
    return pl.pallas_call(
        _tc_block,
        grid=(grid,),
        in_specs=[pl.BlockSpec((rows_per_block, _N), lambda i: (i, 0))],
        out_specs=pl.BlockSpec((rows_per_block, _N), lambda i: (i, 0)),
        out_shape=jax.ShapeDtypeStruct((_TC_ROWS, _N), jnp.float32),
    )(x)


def kernel(X):
    out_sc = _sc_part(X[_TC_ROWS:])
    out_tc = _tc_part(X[:_TC_ROWS])
    return jnp.concatenate([out_tc, out_sc], axis=0)


# speedup vs baseline: 1.4090x; 1.4090x over previous
import jax
import jax.numpy as jnp
from jax import lax
from jax.experimental import pallas as pl
from jax.experimental.pallas import tpu as pltpu
from jax.experimental.pallas import tpu_sc as plsc

_K = 2048
_N = 32768
_B = 128
_L = 16
_NV = _N // _L
_NC = 2
_NS = 16
_NW = _NC * _NS
_RPW = _B // _NW


def _body(x_hbm, out_hbm, xa_v, xb_v, y_v, hist_v, s_v, sem_in, sem_out):
    cid = lax.axis_index("c")
    sid = lax.axis_index("s")
    wid = sid * _NC + cid
    ones16 = jnp.ones((_L,), jnp.int32)
    zero16 = jnp.zeros((_L,), jnp.int32)
    absmask = jnp.int32(0x7FFFFFFF)
    bufs = (xa_v, xb_v)

    @plsc.parallel_loop(0, 2048 // _L, unroll=8)
    def _(t):
        hist_v[pl.ds(t * _L, _L)] = zero16

    in_cp = pltpu.async_copy(x_hbm.at[wid * _RPW], bufs[0], sem_in)
    out_cp = None

    for r in range(_RPW):
        row = wid * _RPW + r
        x_v = bufs[r % 2]
        in_cp.wait()
        if r + 1 < _RPW:
            in_cp = pltpu.async_copy(
                x_hbm.at[row + 1], bufs[(r + 1) % 2], sem_in)

        def u_of(j, x_v=x_v):
            x = x_v[pl.ds(j * _L, _L)]
            return x, lax.bitcast_convert_type(x, jnp.int32) & absmask

        def suffix_pick(nbuckets, rk):
            nch = nbuckets // _L

            @plsc.parallel_loop(
                0, nch, unroll=4, carry=(jnp.int32(0), zero16))
            def carry_out(jj, carry, nch=nch):
                c = nch - 1 - jj
                csum, cnt = carry
                tot = hist_v[pl.ds(c * _L, _L)]
                hist_v[pl.ds(c * _L, _L)] = zero16
                s = plsc.cumsum(lax.rev(tot, dimensions=(0,)))
                s_desc = s + csum
                cnt = cnt + plsc.all_reduce_population_count(s_desc >= rk)
                s_v[pl.ds(c * _L, _L)] = lax.rev(s_desc, dimensions=(0,))
                return csum + jnp.sum(tot), cnt

            _, cntv = carry_out
            b = jnp.max(cntv) - 1
            s_v[pl.ds(nbuckets, _L)] = zero16
            above = jnp.max(plsc.load_gather(s_v, [zero16 + (b + 1)]))
            return b, above

        @plsc.parallel_loop(0, _NV, unroll=8)
        def _(j):
            _, u = u_of(j)
            plsc.addupdate_scatter(hist_v, [u >> 20], ones16)

        b1, above1 = suffix_pick(2048, jnp.int32(_K))
        rk = jnp.int32(_K) - above1
        pref = b1 << 20

        @plsc.parallel_loop(0, _NV, unroll=8)
        def _(j, b1=b1):
            _, u = u_of(j)
            m = (u >> 20) == b1
            plsc.addupdate_scatter(
                hist_v, [(u >> 10) & jnp.int32(1023)], ones16, mask=m)

        b2, above2 = suffix_pick(1024, rk)
        rk = rk - above2
        pref = pref | (b2 << 10)

        @plsc.parallel_loop(0, _NV, unroll=8)
        def _(j, pref=pref):
            _, u = u_of(j)
            m = (u >> 10) == (pref >> 10)
            plsc.addupdate_scatter(
                hist_v, [u & jnp.int32(1023)], ones16, mask=m)

        b3, _ = suffix_pick(1024, rk)
        thr = pref | b3
        if out_cp is not None:
            out_cp.wait()

        @plsc.parallel_loop(0, _NV, unroll=8)
        def _(j, thr=thr):
            x, u = u_of(j)
            y_v[pl.ds(j * _L, _L)] = jnp.where(u >= thr, x, jnp.float32(0.0))

        out_cp = pltpu.async_copy(y_v, out_hbm.at[row], sem_out)

    out_cp.wait()


def kernel(X):
    mesh = plsc.VectorSubcoreMesh(
        core_axis_name="c", subcore_axis_name="s",
        num_cores=_NC, num_subcores=_NS)
    return pl.kernel(
        _body,
        out_type=jax.ShapeDtypeStruct((_B, _N), jnp.float32),
        mesh=mesh,
        scratch_types=[
            pltpu.VMEM((_N,), jnp.float32),
            pltpu.VMEM((_N,), jnp.float32),
            pltpu.VMEM((_N,), jnp.float32),
            pltpu.VMEM((2048,), jnp.int32),
            pltpu.VMEM((2048 + _L,), jnp.int32),
            pltpu.SemaphoreType.DMA,
            pltpu.SemaphoreType.DMA,
        ],
        compiler_params=pltpu.CompilerParams(needs_layout_passes=False),
    )(X)
